# Initial kernel scaffold; baseline (speedup 1.0000x reference)
import functools

import jax
import jax.numpy as jnp
from jax import lax
from jax.experimental import pallas as pl
from jax.experimental.pallas import tpu as pltpu
from jax.experimental.pallas import tpu_sc as plsc


def _table_body(w_ref, w1_ref, b1_ref, out_ref):
    e = jnp.maximum(w_ref[...], 0.0)
    s = jnp.sum(e * w1_ref[...][None, :], axis=1)
    out_ref[...] = jnp.maximum(s + b1_ref[0], 0.0)


def _build_table(w_pad, w1_pad, b1):
    vpad = w_pad.shape[0]
    return pl.pallas_call(
        _table_body,
        out_shape=jax.ShapeDtypeStruct((vpad,), jnp.float32),
        in_specs=[
            pl.BlockSpec(memory_space=pltpu.VMEM),
            pl.BlockSpec(memory_space=pltpu.VMEM),
            pl.BlockSpec(memory_space=pltpu.SMEM),
        ],
        out_specs=pl.BlockSpec(memory_space=pltpu.VMEM),
    )(w_pad, w1_pad, b1)


_SC_INFO = plsc.get_sparse_core_info()
_NC = _SC_INFO.num_cores
_NS = _SC_INFO.num_subcores
_NW = _NC * _NS
_L = _SC_INFO.num_lanes


@functools.cache
def _make_gather(n_flat: int, vpad: int):
    assert n_flat % (_NW * _L) == 0
    b_per_w = n_flat // _NW
    mesh = plsc.VectorSubcoreMesh(core_axis_name="c", subcore_axis_name="s")

    @functools.partial(
        pl.kernel,
        out_type=jax.ShapeDtypeStruct((n_flat,), jnp.float32),
        mesh=mesh,
        scratch_types=[
            pltpu.VMEM((b_per_w,), jnp.int32),
            pltpu.VMEM((b_per_w,), jnp.float32),
            pltpu.VMEM((vpad,), jnp.float32),
        ],
    )
    def gather_kernel(table_hbm, idx_hbm, out_hbm, idx_v, out_v, table_v):
        wid = lax.axis_index("s") * _NC + lax.axis_index("c")
        base = wid * b_per_w
        pltpu.sync_copy(table_hbm, table_v)
        pltpu.sync_copy(idx_hbm.at[pl.ds(base, b_per_w)], idx_v)

        def body(i, carry):
            off = pl.multiple_of(i * _L, _L)
            idx = idx_v[pl.ds(off, _L)]
            out_v[pl.ds(off, _L)] = plsc.load_gather(table_v, [idx])
            return carry

        lax.fori_loop(0, b_per_w // _L, body, 0)
        pltpu.sync_copy(out_v, out_hbm.at[pl.ds(base, b_per_w)])

    return gather_kernel


def kernel(x, W_emb, W1, b1):
    B, Lseq = x.shape
    V, D = W_emb.shape

    vpad = ((V + 127) // 128) * 128
    dpad = ((D + 127) // 128) * 128
    w_pad = jnp.zeros((vpad, dpad), jnp.float32).at[:V, :D].set(W_emb)
    w1_pad = jnp.zeros((dpad,), jnp.float32).at[:D].set(W1[0].astype(jnp.float32))

    table = _build_table(w_pad, w1_pad, b1.astype(jnp.float32))

    x_flat = x.reshape(-1).astype(jnp.int32)
    out_flat = _make_gather(x_flat.shape[0], vpad)(table, x_flat)
    return out_flat.reshape(B, Lseq, 1)


# speedup vs baseline: 42.5278x; 42.5278x over previous
import functools

import jax
import jax.numpy as jnp
from jax import lax
from jax.experimental import pallas as pl
from jax.experimental.pallas import tpu as pltpu
from jax.experimental.pallas import tpu_sc as plsc


def _table_body(w_ref, w1_ref, b1_ref, out_ref):
    e = jnp.maximum(w_ref[...], 0.0)
    s = jnp.sum(e * w1_ref[...][None, :], axis=1)
    out_ref[...] = jnp.maximum(s + b1_ref[0], 0.0)


def _build_table(w_pad, w1_pad, b1):
    vpad = w_pad.shape[0]
    return pl.pallas_call(
        _table_body,
        out_shape=jax.ShapeDtypeStruct((vpad,), jnp.float32),
        in_specs=[
            pl.BlockSpec(memory_space=pltpu.VMEM),
            pl.BlockSpec(memory_space=pltpu.VMEM),
            pl.BlockSpec(memory_space=pltpu.SMEM),
        ],
        out_specs=pl.BlockSpec(memory_space=pltpu.VMEM),
    )(w_pad, w1_pad, b1)


_SC_INFO = plsc.get_sparse_core_info()
_NC = _SC_INFO.num_cores
_NS = _SC_INFO.num_subcores
_NW = _NC * _NS
_L = _SC_INFO.num_lanes


@functools.cache
def _make_gather(n_flat: int, vpad: int):
    assert n_flat % (_NW * _L) == 0
    b_per_w = n_flat // _NW
    mesh = plsc.VectorSubcoreMesh(core_axis_name="c", subcore_axis_name="s")

    @functools.partial(
        pl.kernel,
        out_type=jax.ShapeDtypeStruct((n_flat,), jnp.float32),
        mesh=mesh,
        compiler_params=pltpu.CompilerParams(needs_layout_passes=False),
        scratch_types=[
            pltpu.VMEM((b_per_w,), jnp.int32),
            pltpu.VMEM((b_per_w,), jnp.float32),
            pltpu.VMEM((vpad,), jnp.float32),
        ],
    )
    def gather_kernel(table_hbm, idx_hbm, out_hbm, idx_v, out_v, table_v):
        wid = lax.axis_index("s") * _NC + lax.axis_index("c")
        base = wid * b_per_w
        pltpu.sync_copy(table_hbm, table_v)
        pltpu.sync_copy(idx_hbm.at[pl.ds(base, b_per_w)], idx_v)

        def body(i, carry):
            off = pl.multiple_of(i * _L, _L)
            idx = idx_v[pl.ds(off, _L)]
            out_v[pl.ds(off, _L)] = plsc.load_gather(table_v, [idx])
            return carry

        lax.fori_loop(0, b_per_w // _L, body, 0)
        pltpu.sync_copy(out_v, out_hbm.at[pl.ds(base, b_per_w)])

    return gather_kernel


def kernel(x, W_emb, W1, b1):
    B, Lseq = x.shape
    V, D = W_emb.shape

    vpad = ((V + 127) // 128) * 128
    dpad = ((D + 127) // 128) * 128
    w_pad = jnp.zeros((vpad, dpad), jnp.float32).at[:V, :D].set(W_emb)
    w1_pad = jnp.zeros((dpad,), jnp.float32).at[:D].set(W1[0].astype(jnp.float32))

    table = _build_table(w_pad, w1_pad, b1.astype(jnp.float32))

    x_flat = x.reshape(-1).astype(jnp.int32)
    out_flat = _make_gather(x_flat.shape[0], vpad)(table, x_flat)
    return out_flat.reshape(B, Lseq, 1)


# speedup vs baseline: 63.6935x; 1.4977x over previous
import functools

import jax
import jax.numpy as jnp
from jax import lax
from jax.experimental import pallas as pl
from jax.experimental.pallas import tpu as pltpu
from jax.experimental.pallas import tpu_sc as plsc


def _table_body(w_ref, w1_ref, b1_ref, out_ref):
    e = jnp.maximum(w_ref[...], 0.0)
    s = jnp.sum(e * w1_ref[...], axis=1)
    out_ref[...] = jnp.maximum(s + b1_ref[0], 0.0)


def _build_table(w_emb, w1, b1):
    v = w_emb.shape[0]
    return pl.pallas_call(
        _table_body,
        out_shape=jax.ShapeDtypeStruct((v,), jnp.float32),
        in_specs=[
            pl.BlockSpec(memory_space=pltpu.VMEM),
            pl.BlockSpec(memory_space=pltpu.VMEM),
            pl.BlockSpec(memory_space=pltpu.SMEM),
        ],
        out_specs=pl.BlockSpec(memory_space=pltpu.VMEM),
    )(w_emb, w1, b1)


_SC_INFO = plsc.get_sparse_core_info()
_NC = _SC_INFO.num_cores
_NS = _SC_INFO.num_subcores
_NW = _NC * _NS
_L = _SC_INFO.num_lanes


@functools.cache
def _make_gather(b: int, lseq: int, v: int):
    assert b % _NW == 0
    rows_per_w = b // _NW
    n_chunks = 2
    rows_per_chunk = rows_per_w // n_chunks
    n_per_chunk = rows_per_chunk * lseq
    assert n_per_chunk % _L == 0
    mesh = plsc.VectorSubcoreMesh(core_axis_name="c", subcore_axis_name="s")

    @functools.partial(
        pl.kernel,
        out_type=jax.ShapeDtypeStruct((b, lseq), jnp.float32),
        mesh=mesh,
        compiler_params=pltpu.CompilerParams(needs_layout_passes=False),
        scratch_types=[
            pltpu.VMEM((rows_per_chunk, lseq), jnp.int32),
            pltpu.VMEM((rows_per_chunk, lseq), jnp.float32),
            pltpu.VMEM((v,), jnp.float32),
        ],
    )
    def gather_kernel(table_hbm, x_hbm, out_hbm, idx_v, out_v, table_v):
        wid = lax.axis_index("s") * _NC + lax.axis_index("c")
        r0 = wid * rows_per_w
        pltpu.sync_copy(table_hbm, table_v)

        lanes = lax.iota(jnp.int32, _L)

        for chunk in range(n_chunks):
            rc = r0 + chunk * rows_per_chunk
            pltpu.sync_copy(x_hbm.at[pl.ds(rc, rows_per_chunk)], idx_v)

            @plsc.parallel_loop(0, n_per_chunk // _L, unroll=8)
            def _gather_iter(i):
                p = i.astype(jnp.int32) * _L + lanes
                r = p // lseq
                c = p - r * lseq
                idx = plsc.load_gather(idx_v, [r, c])
                vals = plsc.load_gather(table_v, [idx])
                plsc.store_scatter(out_v, [r, c], vals)

            pltpu.sync_copy(out_v, out_hbm.at[pl.ds(rc, rows_per_chunk)])

    return gather_kernel


def kernel(x, W_emb, W1, b1):
    B, Lseq = x.shape
    V, _ = W_emb.shape
    table = _build_table(
        W_emb.astype(jnp.float32), W1.astype(jnp.float32), b1.astype(jnp.float32)
    )
    out2d = _make_gather(B, Lseq, V)(table, x.astype(jnp.int32))
    return out2d[..., None]


# speedup vs baseline: 77.2681x; 1.2131x over previous
import functools

import jax
import jax.numpy as jnp
from jax import lax
from jax.experimental import pallas as pl
from jax.experimental.pallas import tpu as pltpu
from jax.experimental.pallas import tpu_sc as plsc


def _table_body(wt_ref, w1_ref, b1_ref, out_ref):
    e = jnp.maximum(wt_ref[...], 0.0)
    s = jnp.dot(w1_ref[...], e, preferred_element_type=jnp.float32)
    out_ref[...] = jnp.maximum(s[0] + b1_ref[0], 0.0)


def _build_table(w_emb_t, w1, b1):
    v = w_emb_t.shape[1]
    return pl.pallas_call(
        _table_body,
        out_shape=jax.ShapeDtypeStruct((v,), jnp.float32),
        in_specs=[
            pl.BlockSpec(memory_space=pltpu.VMEM),
            pl.BlockSpec(memory_space=pltpu.VMEM),
            pl.BlockSpec(memory_space=pltpu.SMEM),
        ],
        out_specs=pl.BlockSpec(memory_space=pltpu.VMEM),
    )(w_emb_t, w1, b1)


_SC_INFO = plsc.get_sparse_core_info()
_NC = _SC_INFO.num_cores
_NS = _SC_INFO.num_subcores
_NW = _NC * _NS
_L = _SC_INFO.num_lanes


@functools.cache
def _make_gather(lseq: int, b: int, v: int):
    assert b % (_NW * _L) == 0
    cols_per_w = b // _NW
    mesh = plsc.VectorSubcoreMesh(core_axis_name="c", subcore_axis_name="s")

    @functools.partial(
        pl.kernel,
        out_type=jax.ShapeDtypeStruct((lseq * b,), jnp.float32),
        mesh=mesh,
        compiler_params=pltpu.CompilerParams(needs_layout_passes=False),
        scratch_types=[
            pltpu.VMEM((lseq, cols_per_w), jnp.int32),
            pltpu.VMEM((lseq, cols_per_w), jnp.float32),
            pltpu.VMEM((v,), jnp.float32),
            pltpu.SemaphoreType.DMA,
        ],
    )
    def gather_kernel(table_hbm, xt_hbm, out_hbm, idx_v, out_v, table_v, sem):
        wid = lax.axis_index("s") * _NC + lax.axis_index("c")
        c0 = wid * cols_per_w
        pltpu.sync_copy(table_hbm, table_v)
        pltpu.sync_copy(xt_hbm.at[:, pl.ds(c0, cols_per_w)], idx_v)

        @plsc.parallel_loop(0, cols_per_w // _L, unroll=2)
        def _gather_iter(g):
            off = pl.multiple_of(g.astype(jnp.int32) * _L, _L)
            for r in range(lseq):
                idx = idx_v[r, pl.ds(off, _L)]
                out_v[r, pl.ds(off, _L)] = plsc.load_gather(table_v, [idx])

        copies = [
            pltpu.async_copy(
                out_v.at[r], out_hbm.at[pl.ds(r * b + c0, cols_per_w)], sem
            )
            for r in range(lseq)
        ]
        for cp in copies:
            cp.wait()

    return gather_kernel


def kernel(x, W_emb, W1, b1):
    B, Lseq = x.shape
    V, _ = W_emb.shape
    table = _build_table(
        W_emb.T.astype(jnp.float32), W1.astype(jnp.float32), b1.astype(jnp.float32)
    )
    out_flat = _make_gather(Lseq, B, V)(table, x.T.astype(jnp.int32))
    return out_flat.reshape(Lseq, B).T[..., None]


# speedup vs baseline: 112.0804x; 1.4505x over previous
import functools

import jax
import jax.numpy as jnp
from jax import lax
from jax.experimental import pallas as pl
from jax.experimental.pallas import tpu as pltpu
from jax.experimental.pallas import tpu_sc as plsc


def _table_body(wt_ref, w1_ref, b1_ref, out_ref):
    e = jnp.maximum(wt_ref[...], 0.0)
    s = jnp.dot(w1_ref[...], e, preferred_element_type=jnp.float32)
    out_ref[...] = jnp.maximum(s[0] + b1_ref[0], 0.0)


def _build_table(w_emb_t, w1, b1):
    v = w_emb_t.shape[1]
    return pl.pallas_call(
        _table_body,
        out_shape=jax.ShapeDtypeStruct((v,), jnp.float32),
        in_specs=[
            pl.BlockSpec(memory_space=pltpu.VMEM),
            pl.BlockSpec(memory_space=pltpu.VMEM),
            pl.BlockSpec(memory_space=pltpu.SMEM),
        ],
        out_specs=pl.BlockSpec(memory_space=pltpu.VMEM),
    )(w_emb_t, w1, b1)


_SC_INFO = plsc.get_sparse_core_info()
_NC = _SC_INFO.num_cores
_NS = _SC_INFO.num_subcores
_NW = _NC * _NS
_L = _SC_INFO.num_lanes


@functools.cache
def _make_gather(lseq: int, b: int, v: int):
    assert b % (_NW * _L) == 0
    cols_per_w = b // _NW
    mesh = plsc.VectorSubcoreMesh(core_axis_name="c", subcore_axis_name="s")

    @functools.partial(
        pl.kernel,
        out_type=jax.ShapeDtypeStruct((lseq * b,), jnp.float32),
        mesh=mesh,
        compiler_params=pltpu.CompilerParams(needs_layout_passes=False),
        scratch_types=[
            pltpu.VMEM((lseq, cols_per_w), jnp.int32),
            pltpu.VMEM((lseq, cols_per_w), jnp.float32),
            pltpu.VMEM((v,), jnp.float32),
            pltpu.SemaphoreType.DMA,
        ],
    )
    def gather_kernel(table_hbm, xt_hbm, out_hbm, idx_v, out_v, table_v, sem):
        wid = lax.axis_index("s") * _NC + lax.axis_index("c")
        c0 = wid * cols_per_w
        pltpu.sync_copy(table_hbm, table_v)
        pltpu.sync_copy(xt_hbm.at[:, pl.ds(c0, cols_per_w)], idx_v)

        @plsc.parallel_loop(0, cols_per_w // _L, unroll=2)
        def _gather_iter(g):
            off = pl.multiple_of(g.astype(jnp.int32) * _L, _L)
            for r in range(lseq):
                idx = idx_v[r, pl.ds(off, _L)]
                out_v[r, pl.ds(off, _L)] = plsc.load_gather(table_v, [idx])

        copies = [
            pltpu.async_copy(
                out_v.at[r], out_hbm.at[pl.ds(r * b + c0, cols_per_w)], sem
            )
            for r in range(lseq)
        ]
        for cp in copies:
            cp.wait()

    return gather_kernel


def kernel(x, W_emb, W1, b1):
    B, Lseq = x.shape
    V, _ = W_emb.shape
    table = _build_table(
        W_emb.T.astype(jnp.float32), W1.astype(jnp.float32), b1.astype(jnp.float32)
    )
    out_flat = _make_gather(Lseq, B, V)(table, x.T.astype(jnp.int32))
    return out_flat.reshape(Lseq, 1, B).transpose((2, 0, 1))
